# baseline (speedup 1.0000x reference)
import jax
import jax.numpy as jnp
from jax import lax
from jax.experimental import pallas as pl
from jax.experimental.pallas import tpu as pltpu

B = 64
NB = 30
NSEM = 11
D = 256
GB = 16
RB = GB * NB
F32 = jnp.float32

_WOFF = {'bbd1': 0, 'w1l': 256, 'w2l': 512, 'w1il': 768, 'w1jl': 1024,
         'w2i': 1280, 'w2j': 1536, 'w3i': 1792, 'w3j': 2048,
         'heads': 2304, 'bbd2': 2327}
_WCOLS = 2328
_BOFF = {'bbd1': 0, 'mp1': 256, 'mp2': 512, 'mp3': 768, 'ed1': 1024,
         'heads': 1280, 'ed2': 1303, 'bbd2': 1304}
_BCOLS = 1305
_HOFF = (0, 1, 6, 17, 21, 23)


def _lot_body(z_ref, w_ref, b_ref, out_ref):
    acc = jnp.dot(z_ref[:, :], w_ref[:, :], preferred_element_type=F32)
    out_ref[:, :] = jnp.maximum(acc + b_ref[:, :], 0.0)


def _fused_body(lot2_ref, lot3_ref, wp_ref, rp_ref, bp_ref,
                main_ref, adj_ref,
                s_ref, nbr_ref, x30_ref, af0_ref, apad_ref, bpad_ref):
    j = pl.program_id(0)
    lot = lot2_ref[:, :]

    h = jnp.maximum(jnp.dot(lot, wp_ref[:, _WOFF['bbd1']:_WOFF['bbd1'] + D],
                            preferred_element_type=F32)
                    + bp_ref[:, _BOFF['bbd1']:_BOFF['bbd1'] + D], 0.0)
    ar = (jnp.dot(h, wp_ref[:, _WOFF['bbd2']:_WOFF['bbd2'] + 1],
                  preferred_element_type=F32)
          + bp_ref[:, _BOFF['bbd2']:_BOFF['bbd2'] + 1])
    main_ref[pl.ds(j * RB, RB), 0:1] = ar

    A = jnp.dot(lot, wp_ref[:, _WOFF['w1l']:_WOFF['w1l'] + D],
                preferred_element_type=F32)
    Bt = jnp.dot(lot, wp_ref[:, _WOFF['w2l']:_WOFF['w2l'] + D],
                 preferred_element_type=F32)
    p1 = rp_ref[0:NB, :] + bp_ref[:, _BOFF['ed1']:_BOFF['ed1'] + D]
    p2 = rp_ref[NB:2 * NB, :]
    for i in range(GB):
        apad_ref[32 * i:32 * i + NB, :] = A[NB * i:NB * i + NB, :] + p1
        bpad_ref[32 * i:32 * i + NB, :] = Bt[NB * i:NB * i + NB, :] + p2

    wrow = rp_ref[4 * NB:4 * NB + 1, :][None, :, :]
    eb = bp_ref[0, _BOFF['ed2']]
    for i in range(GB):
        Ai = apad_ref[32 * i:32 * i + NB, :]
        Bi = bpad_ref[32 * i:32 * i + NB, :]
        T = jnp.maximum(Ai[:, None, :] + Bi[None, :, :], 0.0)
        adj_ref[i, :, :] = jnp.sum(T * wrow, axis=-1) + eb

    adjb = jax.nn.sigmoid(adj_ref[:, :, :])
    adj_ref[:, :, :] = adjb
    af = (adjb >= 0.5).astype(F32)
    s_acc = jnp.sum(af, axis=0)
    nbr_acc = None
    for i in range(GB):
        nbr = lax.dot_general(af[i], lot3_ref[i, :, :], (((0,), (0,)), ((), ())),
                              preferred_element_type=F32)
        nbr_acc = nbr if nbr_acc is None else nbr_acc + nbr

    @pl.when(j == 0)
    def _init():
        s_ref[:, :] = s_acc
        nbr_ref[:, :] = nbr_acc
        x30_ref[:, :] = lot3_ref[0, :, :]
        af0_ref[:, :] = af[0]

    @pl.when(j > 0)
    def _acc():
        s_ref[:, :] = s_ref[:, :] + s_acc
        nbr_ref[:, :] = nbr_ref[:, :] + nbr_acc

    @pl.when(j == pl.num_programs(0) - 1)
    def _mp():
        S = s_ref[:, :]
        ones = jnp.ones((NB, 1), F32)
        deg = lax.dot_general(S, ones, (((0,), (0,)), ((), ())),
                              preferred_element_type=F32)
        invd = 1.0 / jnp.where(deg > 0, deg, 1.0)
        mask = deg > 0

        def wcol(name):
            return wp_ref[:, _WOFF[name]:_WOFF[name] + D]

        def brow(name):
            return bp_ref[:, _BOFF[name]:_BOFF[name] + D]

        nbrm = nbr_ref[:, :] * invd
        sm = lax.dot_general(S, rp_ref[3 * NB:4 * NB, :], (((0,), (0,)), ((), ())),
                             preferred_element_type=F32) * invd
        out1 = (jnp.dot(x30_ref[:, :], wcol('w1il'), preferred_element_type=F32)
                + rp_ref[2 * NB:3 * NB, :]
                + jnp.dot(nbrm, wcol('w1jl'), preferred_element_type=F32)
                + sm + brow('mp1'))
        d = jnp.maximum(jnp.where(mask, out1, 0.0), 0.0)

        af0 = af0_ref[:, :]
        for wi, wj, bb in (('w2i', 'w2j', 'mp2'), ('w3i', 'w3j', 'mp3')):
            nbr2 = lax.dot_general(af0, d, (((0,), (0,)), ((), ())),
                                   preferred_element_type=F32) * invd
            out = (jnp.dot(d, wcol(wi), preferred_element_type=F32)
                   + jnp.dot(nbr2, wcol(wj), preferred_element_type=F32)
                   + brow(bb))
            d = jnp.maximum(jnp.where(mask, out, 0.0), 0.0)

        def softmax(x):
            m = jnp.max(x, axis=-1, keepdims=True)
            e = jnp.exp(x - m)
            return e / jnp.sum(e, axis=-1, keepdims=True)

        hw = wp_ref[:, _WOFF['heads']:_WOFF['heads'] + 23]
        hb = bp_ref[:, _BOFF['heads']:_BOFF['heads'] + 23]
        hl = jnp.dot(d, hw, preferred_element_type=F32) + hb
        o = _HOFF
        top = jnp.concatenate([
            jax.nn.sigmoid(hl[:, o[0]:o[1]]),
            hl[:, o[1]:o[2]],
            softmax(hl[:, o[2]:o[3]]),
            hl[:, o[3]:o[4]],
            hl[:, o[4]:o[5]],
        ], axis=1)
        crow = jnp.concatenate([
            jax.nn.sigmoid(hb[:, o[0]:o[1]]),
            hb[:, o[1]:o[2]],
            softmax(hb[:, o[2]:o[3]]),
            hb[:, o[3]:o[4]],
            hb[:, o[4]:o[5]],
        ], axis=1)
        main_ref[:, 1:24] = jnp.broadcast_to(crow, (B * NB, 23))
        main_ref[0:NB, 1:24] = top


def kernel(z, lid_W, lid_b, bbd1_W, bbd1_b, bbd2_W, bbd2_b, ed1_W, ed1_b,
           ed2_W, ed2_b, mp1_W, mp1_b, mp2_W, mp2_b, mp3_W, mp3_b,
           ned_W, ned_b, lh_W, lh_b, bh_W, bh_b, bdh_W, bdh_b, mh_W, mh_b):
    NBLK = 2
    BN = (NB * D) // NBLK
    lot2d = pl.pallas_call(
        _lot_body,
        grid=(NBLK,),
        in_specs=[
            pl.BlockSpec((B, D), lambda j: (0, 0)),
            pl.BlockSpec((D, BN), lambda j: (0, j)),
            pl.BlockSpec((1, BN), lambda j: (0, j)),
        ],
        out_specs=pl.BlockSpec((B, BN), lambda j: (0, j)),
        out_shape=jax.ShapeDtypeStruct((B, NB * D), F32),
    )(z, lid_W, lid_b.reshape(1, NB * D))
    lot = lot2d.reshape(B * NB, D)
    lot3 = lot2d.reshape(B, NB, D)

    wpack = jnp.concatenate([
        bbd1_W,
        ed1_W[:D], ed1_W[D + NB:2 * D + NB],
        mp1_W[:D], mp1_W[D + NB:2 * D + NB],
        mp2_W[:D], mp2_W[D:],
        mp3_W[:D], mp3_W[D:],
        ned_W, bh_W, lh_W, bdh_W, mh_W,
        bbd2_W,
    ], axis=1)
    rpack = jnp.concatenate([
        ed1_W[D:D + NB], ed1_W[2 * D + NB:],
        mp1_W[D:D + NB], mp1_W[2 * D + NB:],
        ed2_W.reshape(1, D), jnp.zeros((1, D), F32),
    ], axis=0)
    bpack = jnp.concatenate([
        bbd1_b, mp1_b, mp2_b, mp3_b, ed1_b,
        ned_b, bh_b, lh_b, bdh_b, mh_b, ed2_b, bbd2_b,
    ])[None, :]

    cmap = lambda *s: pl.BlockSpec(s, lambda j: (0,) * len(s))
    main, adj = pl.pallas_call(
        _fused_body,
        grid=(B // GB,),
        in_specs=[
            pl.BlockSpec((RB, D), lambda j: (j, 0)),
            pl.BlockSpec((GB, NB, D), lambda j: (j, 0, 0)),
            cmap(D, _WCOLS), cmap(122, D), cmap(1, _BCOLS),
        ],
        out_specs=[
            cmap(B * NB, 24),
            pl.BlockSpec((GB, NB, NB), lambda j: (j, 0, 0)),
        ],
        out_shape=[
            jax.ShapeDtypeStruct((B * NB, 24), F32),
            jax.ShapeDtypeStruct((B, NB, NB), F32),
        ],
        scratch_shapes=[
            pltpu.VMEM((NB, NB), F32),
            pltpu.VMEM((NB, D), F32),
            pltpu.VMEM((NB, D), F32),
            pltpu.VMEM((NB, NB), F32),
            pltpu.VMEM((GB * 32, D), F32),
            pltpu.VMEM((GB * 32, D), F32),
        ],
    )(lot, lot3, wpack, rpack, bpack)

    o = _HOFF
    ar = main[:, 0:1]
    lep = main[:, 1 + o[0]:1 + o[1]]
    lg = main[:, 1 + o[1]:1 + o[2]]
    lu = main[:, 1 + o[2]:1 + o[3]]
    lb = main[:, 1 + o[3]:1 + o[4]]
    lm = main[:, 1 + o[4]:1 + o[5]]
    return (lep, lg, lu, lb, lm, adj, ar)


# speedup vs baseline: 1.2421x; 1.2421x over previous
import jax
import jax.numpy as jnp
from jax import lax
from jax.experimental import pallas as pl
from jax.experimental.pallas import tpu as pltpu

B = 64
NB = 30
NSEM = 11
D = 256
GB = 16
RB = GB * NB
F32 = jnp.float32

_BOFF = {'bbd1': 0, 'mp1': 256, 'mp2': 512, 'mp3': 768, 'ed1': 1024,
         'heads': 1280, 'ed2': 1303, 'bbd2': 1304}
_BCOLS = 1305
_HOFF = (0, 1, 6, 17, 21, 23)


def _lot_body(z_ref, w_ref, b_ref, out_ref):
    acc = jnp.dot(z_ref[:, :], w_ref[:, :], preferred_element_type=F32)
    out_ref[:, :] = jnp.maximum(acc + b_ref[:, :], 0.0)


def _fused_body(lot2_ref, lot3_ref, ed1w_ref, ed2r_ref, bw1_ref, bw2_ref,
                mp1_ref, mp2_ref, mp3_ref, hw_ref, bp_ref,
                main_ref, adj_ref,
                s_ref, nbr_ref, x30_ref, af0_ref, apad_ref, bpad_ref):
    j = pl.program_id(0)
    lot = lot2_ref[:, :]

    h = jnp.maximum(jnp.dot(lot, bw1_ref[:, :], preferred_element_type=F32)
                    + bp_ref[:, _BOFF['bbd1']:_BOFF['bbd1'] + D], 0.0)
    ar = (jnp.dot(h, bw2_ref[:, :], preferred_element_type=F32)
          + bp_ref[:, _BOFF['bbd2']:_BOFF['bbd2'] + 1])
    main_ref[pl.ds(j * RB, RB), 0:1] = ar

    A = jnp.dot(lot, ed1w_ref[0:D, :], preferred_element_type=F32)
    Bt = jnp.dot(lot, ed1w_ref[D + NB:2 * D + NB, :], preferred_element_type=F32)
    p1 = ed1w_ref[D:D + NB, :] + bp_ref[:, _BOFF['ed1']:_BOFF['ed1'] + D]
    p2 = ed1w_ref[2 * D + NB:2 * (D + NB), :]
    for i in range(GB):
        apad_ref[32 * i:32 * i + NB, :] = A[NB * i:NB * i + NB, :] + p1
        bpad_ref[32 * i:32 * i + NB, :] = Bt[NB * i:NB * i + NB, :] + p2

    wrow = ed2r_ref[:, :][None, :, :]
    eb = bp_ref[0, _BOFF['ed2']]
    for i in range(GB):
        Ai = apad_ref[32 * i:32 * i + NB, :]
        Bi = bpad_ref[32 * i:32 * i + NB, :]
        T = jnp.maximum(Ai[:, None, :] + Bi[None, :, :], 0.0)
        adj_ref[i, :, :] = jnp.sum(T * wrow, axis=-1) + eb

    adjb = jax.nn.sigmoid(adj_ref[:, :, :])
    adj_ref[:, :, :] = adjb
    af = (adjb >= 0.5).astype(F32)
    s_acc = jnp.sum(af, axis=0)
    nbr_acc = None
    for i in range(GB):
        nbr = lax.dot_general(af[i], lot3_ref[i, :, :], (((0,), (0,)), ((), ())),
                              preferred_element_type=F32)
        nbr_acc = nbr if nbr_acc is None else nbr_acc + nbr

    @pl.when(j == 0)
    def _init():
        s_ref[:, :] = s_acc
        nbr_ref[:, :] = nbr_acc
        x30_ref[:, :] = lot3_ref[0, :, :]
        af0_ref[:, :] = af[0]

    @pl.when(j > 0)
    def _acc():
        s_ref[:, :] = s_ref[:, :] + s_acc
        nbr_ref[:, :] = nbr_ref[:, :] + nbr_acc

    @pl.when(j == pl.num_programs(0) - 1)
    def _mp():
        S = s_ref[:, :]
        ones = jnp.ones((NB, 1), F32)
        deg = lax.dot_general(S, ones, (((0,), (0,)), ((), ())),
                              preferred_element_type=F32)
        invd = 1.0 / jnp.where(deg > 0, deg, 1.0)
        mask = deg > 0

        nbrm = nbr_ref[:, :] * invd
        sm = lax.dot_general(S, mp1_ref[2 * D + NB:2 * (D + NB), :],
                             (((0,), (0,)), ((), ())),
                             preferred_element_type=F32) * invd
        out1 = (jnp.dot(x30_ref[:, :], mp1_ref[0:D, :], preferred_element_type=F32)
                + mp1_ref[D:D + NB, :]
                + jnp.dot(nbrm, mp1_ref[D + NB:2 * D + NB, :],
                          preferred_element_type=F32)
                + sm + bp_ref[:, _BOFF['mp1']:_BOFF['mp1'] + D])
        d = jnp.maximum(jnp.where(mask, out1, 0.0), 0.0)

        af0 = af0_ref[:, :]
        for mp_ref, bname in ((mp2_ref, 'mp2'), (mp3_ref, 'mp3')):
            nbr2 = lax.dot_general(af0, d, (((0,), (0,)), ((), ())),
                                   preferred_element_type=F32) * invd
            out = (jnp.dot(d, mp_ref[0:D, :], preferred_element_type=F32)
                   + jnp.dot(nbr2, mp_ref[D:2 * D, :], preferred_element_type=F32)
                   + bp_ref[:, _BOFF[bname]:_BOFF[bname] + D])
            d = jnp.maximum(jnp.where(mask, out, 0.0), 0.0)

        def softmax(x):
            m = jnp.max(x, axis=-1, keepdims=True)
            e = jnp.exp(x - m)
            return e / jnp.sum(e, axis=-1, keepdims=True)

        hb = bp_ref[:, _BOFF['heads']:_BOFF['heads'] + 23]
        hl = jnp.dot(d, hw_ref[:, :], preferred_element_type=F32) + hb
        o = _HOFF

        def acts(x, n):
            return jnp.concatenate([
                jax.nn.sigmoid(x[:, o[0]:o[1]]),
                x[:, o[1]:o[2]],
                softmax(x[:, o[2]:o[3]]),
                x[:, o[3]:o[4]],
                x[:, o[4]:o[5]],
            ], axis=1)

        main_ref[:, 1:24] = jnp.broadcast_to(acts(hb, 1), (B * NB, 23))
        main_ref[0:NB, 1:24] = acts(hl, NB)


def kernel(z, lid_W, lid_b, bbd1_W, bbd1_b, bbd2_W, bbd2_b, ed1_W, ed1_b,
           ed2_W, ed2_b, mp1_W, mp1_b, mp2_W, mp2_b, mp3_W, mp3_b,
           ned_W, ned_b, lh_W, lh_b, bh_W, bh_b, bdh_W, bdh_b, mh_W, mh_b):
    NBLK = 2
    BN = (NB * D) // NBLK
    lot2d = pl.pallas_call(
        _lot_body,
        grid=(NBLK,),
        in_specs=[
            pl.BlockSpec((B, D), lambda j: (0, 0)),
            pl.BlockSpec((D, BN), lambda j: (0, j)),
            pl.BlockSpec((1, BN), lambda j: (0, j)),
        ],
        out_specs=pl.BlockSpec((B, BN), lambda j: (0, j)),
        out_shape=jax.ShapeDtypeStruct((B, NB * D), F32),
    )(z, lid_W, lid_b.reshape(1, NB * D))
    lot = lot2d.reshape(B * NB, D)
    lot3 = lot2d.reshape(B, NB, D)

    hw = jnp.concatenate([ned_W, bh_W, lh_W, bdh_W, mh_W], axis=1)
    bpack = jnp.concatenate([
        bbd1_b, mp1_b, mp2_b, mp3_b, ed1_b,
        ned_b, bh_b, lh_b, bdh_b, mh_b, ed2_b, bbd2_b,
    ])[None, :]
    ed2row = ed2_W.reshape(1, D)

    cmap = lambda *s: pl.BlockSpec(s, lambda j: (0,) * len(s))
    main, adj = pl.pallas_call(
        _fused_body,
        grid=(B // GB,),
        in_specs=[
            pl.BlockSpec((RB, D), lambda j: (j, 0)),
            pl.BlockSpec((GB, NB, D), lambda j: (j, 0, 0)),
            cmap(2 * (D + NB), D), cmap(1, D), cmap(D, D), cmap(D, 1),
            cmap(2 * (D + NB), D), cmap(2 * D, D), cmap(2 * D, D),
            cmap(D, 23), cmap(1, _BCOLS),
        ],
        out_specs=[
            cmap(B * NB, 24),
            pl.BlockSpec((GB, NB, NB), lambda j: (j, 0, 0)),
        ],
        out_shape=[
            jax.ShapeDtypeStruct((B * NB, 24), F32),
            jax.ShapeDtypeStruct((B, NB, NB), F32),
        ],
        scratch_shapes=[
            pltpu.VMEM((NB, NB), F32),
            pltpu.VMEM((NB, D), F32),
            pltpu.VMEM((NB, D), F32),
            pltpu.VMEM((NB, NB), F32),
            pltpu.VMEM((GB * 32, D), F32),
            pltpu.VMEM((GB * 32, D), F32),
        ],
    )(lot, lot3, ed1_W, ed2row, bbd1_W, bbd2_W, mp1_W, mp2_W, mp3_W, hw, bpack)

    o = _HOFF
    ar = main[:, 0:1]
    lep = main[:, 1 + o[0]:1 + o[1]]
    lg = main[:, 1 + o[1]:1 + o[2]]
    lu = main[:, 1 + o[2]:1 + o[3]]
    lb = main[:, 1 + o[3]:1 + o[4]]
    lm = main[:, 1 + o[4]:1 + o[5]]
    return (lep, lg, lu, lb, lm, adj, ar)


# speedup vs baseline: 1.2657x; 1.0190x over previous
import jax
import jax.numpy as jnp
from jax import lax
from jax.experimental import pallas as pl
from jax.experimental.pallas import tpu as pltpu

B = 64
NB = 30
NSEM = 11
D = 256
GB = 32
RB = GB * NB
F32 = jnp.float32

_BOFF = {'bbd1': 0, 'mp1': 256, 'mp2': 512, 'mp3': 768, 'ed1': 1024,
         'heads': 1280, 'ed2': 1303, 'bbd2': 1304}
_BCOLS = 1305
_HOFF = (0, 1, 6, 17, 21, 23)


def _lot_body(z_ref, w_ref, b_ref, out_ref):
    acc = jnp.dot(z_ref[:, :], w_ref[:, :], preferred_element_type=F32)
    out_ref[:, :] = jnp.maximum(acc + b_ref[:, :], 0.0)


def _fused_body(lot2_ref, lot3_ref, ed1w_ref, ed2r_ref, bw1_ref, bw2_ref,
                mp1_ref, mp2_ref, mp3_ref, hw_ref, bp_ref,
                main_ref, adj_ref,
                s_ref, nbr_ref, x30_ref, af0_ref, apad_ref, bpad_ref):
    j = pl.program_id(0)
    lot = lot2_ref[:, :]

    h = jnp.maximum(jnp.dot(lot, bw1_ref[:, :], preferred_element_type=F32)
                    + bp_ref[:, _BOFF['bbd1']:_BOFF['bbd1'] + D], 0.0)
    ar = (jnp.dot(h, bw2_ref[:, :], preferred_element_type=F32)
          + bp_ref[:, _BOFF['bbd2']:_BOFF['bbd2'] + 1])
    main_ref[pl.ds(j * RB, RB), 0:1] = ar

    A = jnp.dot(lot, ed1w_ref[0:D, :], preferred_element_type=F32)
    Bt = jnp.dot(lot, ed1w_ref[D + NB:2 * D + NB, :], preferred_element_type=F32)
    p1 = ed1w_ref[D:D + NB, :] + bp_ref[:, _BOFF['ed1']:_BOFF['ed1'] + D]
    p2 = ed1w_ref[2 * D + NB:2 * (D + NB), :]
    for i in range(GB):
        apad_ref[32 * i:32 * i + NB, :] = A[NB * i:NB * i + NB, :] + p1
        bpad_ref[32 * i:32 * i + NB, :] = Bt[NB * i:NB * i + NB, :] + p2

    wrow = ed2r_ref[:, :][None, :, :]
    eb = bp_ref[0, _BOFF['ed2']]
    for i in range(GB):
        Ai = apad_ref[32 * i:32 * i + NB, :]
        Bi = bpad_ref[32 * i:32 * i + NB, :]
        T = jnp.maximum(Ai[:, None, :] + Bi[None, :, :], 0.0)
        adj_ref[i, :, :] = jnp.sum(T * wrow, axis=-1) + eb

    adjb = jax.nn.sigmoid(adj_ref[:, :, :])
    adj_ref[:, :, :] = adjb
    af = (adjb >= 0.5).astype(F32)
    s_acc = jnp.sum(af, axis=0)
    nbr_acc = None
    for i in range(GB):
        nbr = lax.dot_general(af[i], lot3_ref[i, :, :], (((0,), (0,)), ((), ())),
                              preferred_element_type=F32)
        nbr_acc = nbr if nbr_acc is None else nbr_acc + nbr

    @pl.when(j == 0)
    def _init():
        s_ref[:, :] = s_acc
        nbr_ref[:, :] = nbr_acc
        x30_ref[:, :] = lot3_ref[0, :, :]
        af0_ref[:, :] = af[0]

    @pl.when(j > 0)
    def _acc():
        s_ref[:, :] = s_ref[:, :] + s_acc
        nbr_ref[:, :] = nbr_ref[:, :] + nbr_acc

    @pl.when(j == pl.num_programs(0) - 1)
    def _mp():
        S = s_ref[:, :]
        ones = jnp.ones((NB, 1), F32)
        deg = lax.dot_general(S, ones, (((0,), (0,)), ((), ())),
                              preferred_element_type=F32)
        invd = 1.0 / jnp.where(deg > 0, deg, 1.0)
        mask = deg > 0

        nbrm = nbr_ref[:, :] * invd
        sm = lax.dot_general(S, mp1_ref[2 * D + NB:2 * (D + NB), :],
                             (((0,), (0,)), ((), ())),
                             preferred_element_type=F32) * invd
        out1 = (jnp.dot(x30_ref[:, :], mp1_ref[0:D, :], preferred_element_type=F32)
                + mp1_ref[D:D + NB, :]
                + jnp.dot(nbrm, mp1_ref[D + NB:2 * D + NB, :],
                          preferred_element_type=F32)
                + sm + bp_ref[:, _BOFF['mp1']:_BOFF['mp1'] + D])
        d = jnp.maximum(jnp.where(mask, out1, 0.0), 0.0)

        af0 = af0_ref[:, :]
        for mp_ref, bname in ((mp2_ref, 'mp2'), (mp3_ref, 'mp3')):
            nbr2 = lax.dot_general(af0, d, (((0,), (0,)), ((), ())),
                                   preferred_element_type=F32) * invd
            out = (jnp.dot(d, mp_ref[0:D, :], preferred_element_type=F32)
                   + jnp.dot(nbr2, mp_ref[D:2 * D, :], preferred_element_type=F32)
                   + bp_ref[:, _BOFF[bname]:_BOFF[bname] + D])
            d = jnp.maximum(jnp.where(mask, out, 0.0), 0.0)

        def softmax(x):
            m = jnp.max(x, axis=-1, keepdims=True)
            e = jnp.exp(x - m)
            return e / jnp.sum(e, axis=-1, keepdims=True)

        hb = bp_ref[:, _BOFF['heads']:_BOFF['heads'] + 23]
        hl = jnp.dot(d, hw_ref[:, :], preferred_element_type=F32) + hb
        o = _HOFF

        def acts(x, n):
            return jnp.concatenate([
                jax.nn.sigmoid(x[:, o[0]:o[1]]),
                x[:, o[1]:o[2]],
                softmax(x[:, o[2]:o[3]]),
                x[:, o[3]:o[4]],
                x[:, o[4]:o[5]],
            ], axis=1)

        main_ref[:, 1:24] = jnp.broadcast_to(acts(hb, 1), (B * NB, 23))
        main_ref[0:NB, 1:24] = acts(hl, NB)


def kernel(z, lid_W, lid_b, bbd1_W, bbd1_b, bbd2_W, bbd2_b, ed1_W, ed1_b,
           ed2_W, ed2_b, mp1_W, mp1_b, mp2_W, mp2_b, mp3_W, mp3_b,
           ned_W, ned_b, lh_W, lh_b, bh_W, bh_b, bdh_W, bdh_b, mh_W, mh_b):
    NBLK = 2
    BN = (NB * D) // NBLK
    lot2d = pl.pallas_call(
        _lot_body,
        grid=(NBLK,),
        in_specs=[
            pl.BlockSpec((B, D), lambda j: (0, 0)),
            pl.BlockSpec((D, BN), lambda j: (0, j)),
            pl.BlockSpec((1, BN), lambda j: (0, j)),
        ],
        out_specs=pl.BlockSpec((B, BN), lambda j: (0, j)),
        out_shape=jax.ShapeDtypeStruct((B, NB * D), F32),
    )(z, lid_W, lid_b.reshape(1, NB * D))
    lot = lot2d.reshape(B * NB, D)
    lot3 = lot2d.reshape(B, NB, D)

    hw = jnp.concatenate([ned_W, bh_W, lh_W, bdh_W, mh_W], axis=1)
    bpack = jnp.concatenate([
        bbd1_b, mp1_b, mp2_b, mp3_b, ed1_b,
        ned_b, bh_b, lh_b, bdh_b, mh_b, ed2_b, bbd2_b,
    ])[None, :]
    ed2row = ed2_W.reshape(1, D)

    cmap = lambda *s: pl.BlockSpec(s, lambda j: (0,) * len(s))
    main, adj = pl.pallas_call(
        _fused_body,
        grid=(B // GB,),
        in_specs=[
            pl.BlockSpec((RB, D), lambda j: (j, 0)),
            pl.BlockSpec((GB, NB, D), lambda j: (j, 0, 0)),
            cmap(2 * (D + NB), D), cmap(1, D), cmap(D, D), cmap(D, 1),
            cmap(2 * (D + NB), D), cmap(2 * D, D), cmap(2 * D, D),
            cmap(D, 23), cmap(1, _BCOLS),
        ],
        out_specs=[
            cmap(B * NB, 24),
            pl.BlockSpec((GB, NB, NB), lambda j: (j, 0, 0)),
        ],
        out_shape=[
            jax.ShapeDtypeStruct((B * NB, 24), F32),
            jax.ShapeDtypeStruct((B, NB, NB), F32),
        ],
        scratch_shapes=[
            pltpu.VMEM((NB, NB), F32),
            pltpu.VMEM((NB, D), F32),
            pltpu.VMEM((NB, D), F32),
            pltpu.VMEM((NB, NB), F32),
            pltpu.VMEM((GB * 32, D), F32),
            pltpu.VMEM((GB * 32, D), F32),
        ],
    )(lot, lot3, ed1_W, ed2row, bbd1_W, bbd2_W, mp1_W, mp2_W, mp3_W, hw, bpack)

    o = _HOFF
    ar = main[:, 0:1]
    lep = main[:, 1 + o[0]:1 + o[1]]
    lg = main[:, 1 + o[1]:1 + o[2]]
    lu = main[:, 1 + o[2]:1 + o[3]]
    lb = main[:, 1 + o[3]:1 + o[4]]
    lm = main[:, 1 + o[4]:1 + o[5]]
    return (lep, lg, lu, lb, lm, adj, ar)


# speedup vs baseline: 1.2835x; 1.0141x over previous
import jax
import jax.numpy as jnp
from jax import lax
from jax.experimental import pallas as pl
from jax.experimental.pallas import tpu as pltpu

B = 64
NB = 30
NSEM = 11
D = 256
GB = 32
RB = GB * NB
F32 = jnp.float32

_BOFF = {'bbd1': 0, 'mp1': 256, 'mp2': 512, 'mp3': 768, 'ed1': 1024,
         'heads': 1280, 'ed2': 1303, 'bbd2': 1304}
_BCOLS = 1305
_HOFF = (0, 1, 6, 17, 21, 23)


def _lot_body(z_ref, w_ref, b_ref, out_ref):
    acc = jnp.dot(z_ref[:, :], w_ref[:, :], preferred_element_type=F32)
    out_ref[:, :] = jnp.maximum(acc + b_ref[:, :], 0.0)


def _fused_body(lot2_ref, lot3_ref, ed1w_ref, ed2r_ref, bw1_ref, bw2_ref,
                mp1_ref, mp2_ref, mp3_ref, hw_ref, bp_ref,
                main_ref, adj_ref,
                s_ref, nbr_ref, x30_ref, af0_ref, apad_ref, bpad_ref):
    j = pl.program_id(0)
    lot = lot2_ref[:, :]

    h = jnp.maximum(jnp.dot(lot, bw1_ref[:, :], preferred_element_type=F32)
                    + bp_ref[:, _BOFF['bbd1']:_BOFF['bbd1'] + D], 0.0)
    ar = (jnp.dot(h, bw2_ref[:, :], preferred_element_type=F32)
          + bp_ref[:, _BOFF['bbd2']:_BOFF['bbd2'] + 1])
    main_ref[pl.ds(j * RB, RB), 0:1] = ar

    A = jnp.dot(lot, ed1w_ref[0:D, :], preferred_element_type=F32)
    Bt = jnp.dot(lot, ed1w_ref[D + NB:2 * D + NB, :], preferred_element_type=F32)
    p1 = ed1w_ref[D:D + NB, :] + bp_ref[:, _BOFF['ed1']:_BOFF['ed1'] + D]
    p2 = ed1w_ref[2 * D + NB:2 * (D + NB), :]
    for i in range(GB):
        apad_ref[32 * i:32 * i + NB, :] = A[NB * i:NB * i + NB, :] + p1
        bpad_ref[32 * i:32 * i + NB, :] = Bt[NB * i:NB * i + NB, :] + p2

    wcol = ed2r_ref[:, :]
    eb = bp_ref[0, _BOFF['ed2']]
    for i in range(GB):
        Ai = apad_ref[32 * i:32 * (i + 1), :]
        Bi = bpad_ref[32 * i:32 * (i + 1), :]
        T = jnp.maximum(Ai[:, None, :] + Bi[None, :, :], 0.0)
        lcol = jnp.dot(T.reshape(32 * 32, D), wcol,
                       preferred_element_type=F32)
        adj_ref[i, :, :] = lcol.reshape(32, 32)[:NB, :NB] + eb

    adjb = jax.nn.sigmoid(adj_ref[:, :, :])
    adj_ref[:, :, :] = adjb
    af = (adjb >= 0.5).astype(F32)
    s_acc = jnp.sum(af, axis=0)
    nbr_acc = None
    for i in range(GB):
        nbr = lax.dot_general(af[i], lot3_ref[i, :, :], (((0,), (0,)), ((), ())),
                              preferred_element_type=F32)
        nbr_acc = nbr if nbr_acc is None else nbr_acc + nbr

    @pl.when(j == 0)
    def _init():
        s_ref[:, :] = s_acc
        nbr_ref[:, :] = nbr_acc
        x30_ref[:, :] = lot3_ref[0, :, :]
        af0_ref[:, :] = af[0]

    @pl.when(j > 0)
    def _acc():
        s_ref[:, :] = s_ref[:, :] + s_acc
        nbr_ref[:, :] = nbr_ref[:, :] + nbr_acc

    @pl.when(j == pl.num_programs(0) - 1)
    def _mp():
        S = s_ref[:, :]
        ones = jnp.ones((NB, 1), F32)
        deg = lax.dot_general(S, ones, (((0,), (0,)), ((), ())),
                              preferred_element_type=F32)
        invd = 1.0 / jnp.where(deg > 0, deg, 1.0)
        mask = deg > 0

        nbrm = nbr_ref[:, :] * invd
        sm = lax.dot_general(S, mp1_ref[2 * D + NB:2 * (D + NB), :],
                             (((0,), (0,)), ((), ())),
                             preferred_element_type=F32) * invd
        out1 = (jnp.dot(x30_ref[:, :], mp1_ref[0:D, :], preferred_element_type=F32)
                + mp1_ref[D:D + NB, :]
                + jnp.dot(nbrm, mp1_ref[D + NB:2 * D + NB, :],
                          preferred_element_type=F32)
                + sm + bp_ref[:, _BOFF['mp1']:_BOFF['mp1'] + D])
        d = jnp.maximum(jnp.where(mask, out1, 0.0), 0.0)

        af0 = af0_ref[:, :]
        for mp_ref, bname in ((mp2_ref, 'mp2'), (mp3_ref, 'mp3')):
            nbr2 = lax.dot_general(af0, d, (((0,), (0,)), ((), ())),
                                   preferred_element_type=F32) * invd
            out = (jnp.dot(d, mp_ref[0:D, :], preferred_element_type=F32)
                   + jnp.dot(nbr2, mp_ref[D:2 * D, :], preferred_element_type=F32)
                   + bp_ref[:, _BOFF[bname]:_BOFF[bname] + D])
            d = jnp.maximum(jnp.where(mask, out, 0.0), 0.0)

        def softmax(x):
            m = jnp.max(x, axis=-1, keepdims=True)
            e = jnp.exp(x - m)
            return e / jnp.sum(e, axis=-1, keepdims=True)

        hb = bp_ref[:, _BOFF['heads']:_BOFF['heads'] + 23]
        hl = jnp.dot(d, hw_ref[:, :], preferred_element_type=F32) + hb
        o = _HOFF

        def acts(x, n):
            return jnp.concatenate([
                jax.nn.sigmoid(x[:, o[0]:o[1]]),
                x[:, o[1]:o[2]],
                softmax(x[:, o[2]:o[3]]),
                x[:, o[3]:o[4]],
                x[:, o[4]:o[5]],
            ], axis=1)

        main_ref[:, 1:24] = jnp.broadcast_to(acts(hb, 1), (B * NB, 23))
        main_ref[0:NB, 1:24] = acts(hl, NB)


def kernel(z, lid_W, lid_b, bbd1_W, bbd1_b, bbd2_W, bbd2_b, ed1_W, ed1_b,
           ed2_W, ed2_b, mp1_W, mp1_b, mp2_W, mp2_b, mp3_W, mp3_b,
           ned_W, ned_b, lh_W, lh_b, bh_W, bh_b, bdh_W, bdh_b, mh_W, mh_b):
    NBLK = 2
    BN = (NB * D) // NBLK
    lot2d = pl.pallas_call(
        _lot_body,
        grid=(NBLK,),
        in_specs=[
            pl.BlockSpec((B, D), lambda j: (0, 0)),
            pl.BlockSpec((D, BN), lambda j: (0, j)),
            pl.BlockSpec((1, BN), lambda j: (0, j)),
        ],
        out_specs=pl.BlockSpec((B, BN), lambda j: (0, j)),
        out_shape=jax.ShapeDtypeStruct((B, NB * D), F32),
    )(z, lid_W, lid_b.reshape(1, NB * D))
    lot = lot2d.reshape(B * NB, D)
    lot3 = lot2d.reshape(B, NB, D)

    hw = jnp.concatenate([ned_W, bh_W, lh_W, bdh_W, mh_W], axis=1)
    bpack = jnp.concatenate([
        bbd1_b, mp1_b, mp2_b, mp3_b, ed1_b,
        ned_b, bh_b, lh_b, bdh_b, mh_b, ed2_b, bbd2_b,
    ])[None, :]

    cmap = lambda *s: pl.BlockSpec(s, lambda j: (0,) * len(s))
    main, adj = pl.pallas_call(
        _fused_body,
        grid=(B // GB,),
        in_specs=[
            pl.BlockSpec((RB, D), lambda j: (j, 0)),
            pl.BlockSpec((GB, NB, D), lambda j: (j, 0, 0)),
            cmap(2 * (D + NB), D), cmap(D, 1), cmap(D, D), cmap(D, 1),
            cmap(2 * (D + NB), D), cmap(2 * D, D), cmap(2 * D, D),
            cmap(D, 23), cmap(1, _BCOLS),
        ],
        out_specs=[
            cmap(B * NB, 24),
            pl.BlockSpec((GB, NB, NB), lambda j: (j, 0, 0)),
        ],
        out_shape=[
            jax.ShapeDtypeStruct((B * NB, 24), F32),
            jax.ShapeDtypeStruct((B, NB, NB), F32),
        ],
        scratch_shapes=[
            pltpu.VMEM((NB, NB), F32),
            pltpu.VMEM((NB, D), F32),
            pltpu.VMEM((NB, D), F32),
            pltpu.VMEM((NB, NB), F32),
            pltpu.VMEM((GB * 32, D), F32),
            pltpu.VMEM((GB * 32, D), F32),
        ],
    )(lot, lot3, ed1_W, ed2_W, bbd1_W, bbd2_W, mp1_W, mp2_W, mp3_W, hw, bpack)

    o = _HOFF
    ar = main[:, 0:1]
    lep = main[:, 1 + o[0]:1 + o[1]]
    lg = main[:, 1 + o[1]:1 + o[2]]
    lu = main[:, 1 + o[2]:1 + o[3]]
    lb = main[:, 1 + o[3]:1 + o[4]]
    lm = main[:, 1 + o[4]:1 + o[5]]
    return (lep, lg, lu, lb, lm, adj, ar)
